# baseline (speedup 1.0000x reference)
import functools
import jax
import jax.numpy as jnp
from jax import lax
from jax.experimental import pallas as pl
from jax.experimental.pallas import tpu as pltpu
from jax.experimental.pallas import tpu_sc as plsc

_FREQ_MASK_PARAM = 27
_TIME_MASK_PARAM = 100
_N_FREQ_MASKS = 2
_N_TIME_MASKS = 2
_TIME_MASK_RATIO = 0.05

_CHUNK = 120
_NSLOT = 2
_NW = 32


def _sc_body(x_hbm, fk_hbm, tk_hbm, o_hbm, in_buf, out_buf, fk_v, tk_v,
             in_sem, out_sem):
    B, T, F = x_hbm.shape
    nv = F // 16
    nchunks = T // _CHUNK
    wid = lax.axis_index("s") * 2 + lax.axis_index("c")
    per_w = B // _NW

    def in_copy(b, ci, sl):
        return pltpu.make_async_copy(
            x_hbm.at[b, pl.ds(ci * _CHUNK, _CHUNK)], in_buf.at[sl],
            in_sem.at[sl])

    def out_copy(b, ci, sl):
        return pltpu.make_async_copy(
            out_buf.at[sl], o_hbm.at[b, pl.ds(ci * _CHUNK, _CHUNK)],
            out_sem.at[sl])

    for i in range(per_w):
        b = wid * per_w + i
        pltpu.sync_copy(fk_hbm.at[b], fk_v)
        pltpu.sync_copy(tk_hbm.at[b], tk_v)
        fks = [fk_v[0, pl.ds(j * 16, 16)] for j in range(nv)]

        for sl in range(_NSLOT):
            in_copy(b, sl, sl).start()

        for ci in range(nchunks):
            sl = ci % _NSLOT
            in_copy(b, ci, sl).wait()
            if ci >= _NSLOT:
                out_copy(b, ci - _NSLOT, sl).wait()

            def row_step(r, carry, sl=sl, fks=fks, base=ci * _CHUNK):
                rg = base + r
                tf = tk_v[rg // 8, pl.ds((rg % 8) * 16, 16)]
                for j in range(nv):
                    v = in_buf[sl, r, pl.ds(j * 16, 16)]
                    out_buf[sl, r, pl.ds(j * 16, 16)] = v * fks[j] * tf
                return carry

            lax.fori_loop(0, _CHUNK, row_step, 0)
            out_copy(b, ci, sl).start()
            if ci + _NSLOT < nchunks:
                in_copy(b, ci + _NSLOT, sl).start()

        for ci in range(nchunks - _NSLOT, nchunks):
            out_copy(b, ci, ci % _NSLOT).wait()


def _keep_factors(B, T, F):
    key = jax.random.key(42)
    col = jnp.arange(F, dtype=jnp.int32)[None, :]
    fkeep = jnp.ones((B, F), jnp.float32)
    for _ in range(_N_FREQ_MASKS):
        key, k1, k2 = jax.random.split(key, 3)
        f = jax.random.randint(k1, (B,), 0, _FREQ_MASK_PARAM + 1)
        f0 = jax.random.randint(k2, (B,), 0, max(1, F - _FREQ_MASK_PARAM))
        fkeep = fkeep * ((col < f0[:, None]) | (col >= (f0 + f)[:, None]))
    t_max = _TIME_MASK_PARAM
    if _TIME_MASK_RATIO is not None:
        t_max = min(t_max, int(_TIME_MASK_RATIO * T))
    row = jnp.arange(T, dtype=jnp.int32)[None, :]
    tkeep = jnp.ones((B, T), jnp.float32)
    for _ in range(_N_TIME_MASKS):
        key, k1, k2 = jax.random.split(key, 3)
        t = jax.random.randint(k1, (B,), 0, max(1, t_max + 1))
        t0 = jax.random.randint(k2, (B,), 0, max(1, T - t_max))
        tkeep = tkeep * ((row < t0[:, None]) | (row >= (t0 + t)[:, None]))
    return fkeep, tkeep


def kernel(x):
    B, T, F = x.shape
    fkeep, tkeep = _keep_factors(B, T, F)
    mesh = plsc.VectorSubcoreMesh(core_axis_name="c", subcore_axis_name="s")
    run = pl.kernel(
        _sc_body,
        out_type=jax.ShapeDtypeStruct((B, T, F), x.dtype),
        mesh=mesh,
        scratch_types=[
            pltpu.VMEM((_NSLOT, _CHUNK, F), jnp.float32),
            pltpu.VMEM((_NSLOT, _CHUNK, F), jnp.float32),
            pltpu.VMEM((1, F), jnp.float32),
            pltpu.VMEM((T // 8, 128), jnp.float32),
            pltpu.SemaphoreType.DMA((_NSLOT,)),
            pltpu.SemaphoreType.DMA((_NSLOT,)),
        ],
    )
    tk128 = jnp.repeat(tkeep.reshape(B, T // 8, 8), 16, axis=2)
    return run(x, fkeep[:, None, :], tk128)


# speedup vs baseline: 1.0162x; 1.0162x over previous
import functools
import jax
import jax.numpy as jnp
from jax import lax
from jax.experimental import pallas as pl
from jax.experimental.pallas import tpu as pltpu
from jax.experimental.pallas import tpu_sc as plsc

_FREQ_MASK_PARAM = 27
_TIME_MASK_PARAM = 100
_N_FREQ_MASKS = 2
_N_TIME_MASKS = 2
_TIME_MASK_RATIO = 0.05

_CHUNK = 120
_NSLOT = 2
_NW = 32


def _sc_body(x_hbm, fk_hbm, tk_hbm, o_hbm, in_buf, out_buf, fk_v, tk_v,
             in_sem, out_sem):
    B, T, F = x_hbm.shape
    nv = F // 16
    nchunks = T // _CHUNK
    wid = lax.axis_index("s") * 2 + lax.axis_index("c")
    per_w = B // _NW

    def in_copy(b, ci, sl):
        return pltpu.make_async_copy(
            x_hbm.at[b, pl.ds(ci * _CHUNK, _CHUNK)], in_buf.at[sl],
            in_sem.at[sl])

    def out_copy(b, ci, sl):
        return pltpu.make_async_copy(
            out_buf.at[sl], o_hbm.at[b, pl.ds(ci * _CHUNK, _CHUNK)],
            out_sem.at[sl])

    for i in range(per_w):
        b = wid * per_w + i
        pltpu.sync_copy(fk_hbm.at[b], fk_v)
        pltpu.sync_copy(tk_hbm.at[b], tk_v)
        fks = [fk_v[0, pl.ds(j * 16, 16)] for j in range(nv)]

        for sl in range(_NSLOT):
            in_copy(b, sl, sl).start()

        def chunk_step(ci, carry, b=b, fks=fks):
            sl = lax.rem(ci, _NSLOT)
            in_copy(b, ci, sl).wait()

            @pl.when(ci >= _NSLOT)
            def _():
                out_copy(b, ci - _NSLOT, sl).wait()

            def grp_step(g, carry2, sl=sl, fks=fks):
                r8 = g * 8
                for k in range(8):
                    tf = tk_v[ci * (_CHUNK // 8) + g, pl.ds(k * 16, 16)]
                    for j in range(nv):
                        v = in_buf[sl, r8 + k, pl.ds(j * 16, 16)]
                        out_buf[sl, r8 + k, pl.ds(j * 16, 16)] = v * fks[j] * tf
                return carry2

            lax.fori_loop(0, _CHUNK // 8, grp_step, 0)
            out_copy(b, ci, sl).start()

            @pl.when(ci + _NSLOT < nchunks)
            def _():
                in_copy(b, ci + _NSLOT, sl).start()

            return carry

        lax.fori_loop(0, nchunks, chunk_step, 0)
        for ci in range(nchunks - _NSLOT, nchunks):
            out_copy(b, ci, ci % _NSLOT).wait()


def _keep_factors(B, T, F):
    key = jax.random.key(42)
    col = jnp.arange(F, dtype=jnp.int32)[None, :]
    fkeep = jnp.ones((B, F), jnp.float32)
    for _ in range(_N_FREQ_MASKS):
        key, k1, k2 = jax.random.split(key, 3)
        f = jax.random.randint(k1, (B,), 0, _FREQ_MASK_PARAM + 1)
        f0 = jax.random.randint(k2, (B,), 0, max(1, F - _FREQ_MASK_PARAM))
        fkeep = fkeep * ((col < f0[:, None]) | (col >= (f0 + f)[:, None]))
    t_max = _TIME_MASK_PARAM
    if _TIME_MASK_RATIO is not None:
        t_max = min(t_max, int(_TIME_MASK_RATIO * T))
    row = jnp.arange(T, dtype=jnp.int32)[None, :]
    tkeep = jnp.ones((B, T), jnp.float32)
    for _ in range(_N_TIME_MASKS):
        key, k1, k2 = jax.random.split(key, 3)
        t = jax.random.randint(k1, (B,), 0, max(1, t_max + 1))
        t0 = jax.random.randint(k2, (B,), 0, max(1, T - t_max))
        tkeep = tkeep * ((row < t0[:, None]) | (row >= (t0 + t)[:, None]))
    return fkeep, tkeep


def kernel(x):
    B, T, F = x.shape
    fkeep, tkeep = _keep_factors(B, T, F)
    mesh = plsc.VectorSubcoreMesh(core_axis_name="c", subcore_axis_name="s")
    run = pl.kernel(
        _sc_body,
        out_type=jax.ShapeDtypeStruct((B, T, F), x.dtype),
        mesh=mesh,
        scratch_types=[
            pltpu.VMEM((_NSLOT, _CHUNK, F), jnp.float32),
            pltpu.VMEM((_NSLOT, _CHUNK, F), jnp.float32),
            pltpu.VMEM((1, F), jnp.float32),
            pltpu.VMEM((T // 8, 128), jnp.float32),
            pltpu.SemaphoreType.DMA((_NSLOT,)),
            pltpu.SemaphoreType.DMA((_NSLOT,)),
        ],
    )
    tk128 = jnp.repeat(tkeep.reshape(B, T // 8, 8), 16, axis=2)
    return run(x, fkeep[:, None, :], tk128)
